# baseline (speedup 1.0000x reference)
import functools

import jax
import jax.numpy as jnp
from jax.experimental import pallas as pl
from jax.experimental.pallas import tpu as pltpu

_LOG2E = 1.4426950408889634


def _attn_body(n_blocks, k_blocks, bs, dh, scale, halves,
               bi_ref, q_ref, k_ref, v_ref, o_ref, kb_ref, vb_ref):
    h2 = pl.program_id(1)
    nb_half = n_blocks // halves

    @pl.when(h2 == 0)
    def _cast_kv():
        kb_ref[...] = k_ref[0].astype(jnp.bfloat16)
        vb_ref[:, :dh] = v_ref[0].astype(jnp.bfloat16)

    @pl.when((pl.program_id(0) == 0) & (h2 == 0))
    def _init_ones():
        vb_ref[:, dh:] = jnp.ones((n_blocks * bs, dh), jnp.bfloat16)

    for t in range(nb_half):
        n = h2 * nb_half + t
        q = (q_ref[0, pl.ds(t * bs, bs), :] * (scale * _LOG2E)
             ).astype(jnp.bfloat16)
        kg = []
        vg = []
        for j in range(k_blocks):
            idx = bi_ref[n * k_blocks + j]
            kg.append(kb_ref[pl.ds(idx * bs, bs), :])
            vg.append(vb_ref[pl.ds(idx * bs, bs), :])
        kg = jnp.concatenate(kg, axis=0)
        vg = jnp.concatenate(vg, axis=0)
        s = jax.lax.dot_general(q, kg, (((1,), (1,)), ((), ())),
                                preferred_element_type=jnp.float32)
        e = jnp.exp2(s).astype(jnp.bfloat16)
        ud = jax.lax.dot_general(e, vg, (((1,), (0,)), ((), ())),
                                 preferred_element_type=jnp.float32)
        o_ref[0, pl.ds(t * bs, bs), :] = ud[:, :dh] / ud[:, dh:]


def kernel(query, key, value, block_index):
    B, H, S, Dh = query.shape
    n_blocks, k_blocks = block_index.shape
    bs = S // n_blocks
    BH = B * H
    scale = 1.0 / float(Dh) ** 0.5
    halves = 2
    Sh = S // halves

    q3 = query.reshape(BH, S, Dh)
    k3 = key.reshape(BH, S, Dh)
    v3 = value.reshape(BH, S, Dh)
    bi = block_index.reshape(-1).astype(jnp.int32)

    body = functools.partial(_attn_body, n_blocks, k_blocks, bs, Dh, scale,
                             halves)
    out = pl.pallas_call(
        body,
        grid_spec=pltpu.PrefetchScalarGridSpec(
            num_scalar_prefetch=1,
            grid=(BH, halves),
            in_specs=[
                pl.BlockSpec((1, Sh, Dh), lambda bh, h2, bi_ref: (bh, h2, 0)),
                pl.BlockSpec((1, S, Dh), lambda bh, h2, bi_ref: (bh, 0, 0)),
                pl.BlockSpec((1, S, Dh), lambda bh, h2, bi_ref: (bh, 0, 0)),
            ],
            out_specs=pl.BlockSpec((1, Sh, Dh), lambda bh, h2, bi_ref: (bh, h2, 0)),
            scratch_shapes=[
                pltpu.VMEM((S, Dh), jnp.bfloat16),
                pltpu.VMEM((S, 2 * Dh), jnp.bfloat16),
            ],
        ),
        out_shape=jax.ShapeDtypeStruct((BH, S, Dh), jnp.float32),
    )(bi, q3, k3, v3)
    return out.reshape(B, H, S, Dh)


# speedup vs baseline: 1.5470x; 1.5470x over previous
import functools

import jax
import jax.numpy as jnp
from jax.experimental import pallas as pl
from jax.experimental.pallas import tpu as pltpu

_LOG2E = 1.4426950408889634


def _attn_body(n_blocks, k_blocks, bs, dh, scale,
               bi_ref, q_ref, k_ref, v_ref, o_ref, kb_ref, vb_ref):
    kb_ref[...] = k_ref[0].astype(jnp.bfloat16)
    vb_ref[:, :dh] = v_ref[0].astype(jnp.bfloat16)

    @pl.when(pl.program_id(0) == 0)
    def _init_ones():
        vb_ref[:, dh:] = jnp.ones((n_blocks * bs, dh), jnp.bfloat16)

    for n in range(n_blocks):
        q = (q_ref[0, pl.ds(n * bs, bs), :] * (scale * _LOG2E)
             ).astype(jnp.bfloat16)
        kg = []
        vg = []
        for j in range(k_blocks):
            idx = bi_ref[n * k_blocks + j]
            kg.append(kb_ref[pl.ds(idx * bs, bs), :])
            vg.append(vb_ref[pl.ds(idx * bs, bs), :])
        kg = jnp.concatenate(kg, axis=0)
        vg = jnp.concatenate(vg, axis=0)
        s = jax.lax.dot_general(q, kg, (((1,), (1,)), ((), ())),
                                preferred_element_type=jnp.float32)
        e = jnp.exp2(s).astype(jnp.bfloat16)
        ud = jax.lax.dot_general(e, vg, (((1,), (0,)), ((), ())),
                                 preferred_element_type=jnp.float32)
        o_ref[0, pl.ds(n * bs, bs), :] = ud[:, :dh] / ud[:, dh:]


def kernel(query, key, value, block_index):
    B, H, S, Dh = query.shape
    n_blocks, k_blocks = block_index.shape
    bs = S // n_blocks
    BH = B * H
    scale = 1.0 / float(Dh) ** 0.5

    q3 = query.reshape(BH, S, Dh)
    k3 = key.reshape(BH, S, Dh)
    v3 = value.reshape(BH, S, Dh)
    bi = block_index.reshape(-1).astype(jnp.int32)

    body = functools.partial(_attn_body, n_blocks, k_blocks, bs, Dh, scale)
    out = pl.pallas_call(
        body,
        grid_spec=pltpu.PrefetchScalarGridSpec(
            num_scalar_prefetch=1,
            grid=(BH,),
            in_specs=[
                pl.BlockSpec((1, S, Dh), lambda bh, bi_ref: (bh, 0, 0)),
                pl.BlockSpec((1, S, Dh), lambda bh, bi_ref: (bh, 0, 0)),
                pl.BlockSpec((1, S, Dh), lambda bh, bi_ref: (bh, 0, 0)),
            ],
            out_specs=pl.BlockSpec((1, S, Dh), lambda bh, bi_ref: (bh, 0, 0)),
            scratch_shapes=[
                pltpu.VMEM((S, Dh), jnp.bfloat16),
                pltpu.VMEM((S, 2 * Dh), jnp.bfloat16),
            ],
        ),
        out_shape=jax.ShapeDtypeStruct((BH, S, Dh), jnp.float32),
    )(bi, q3, k3, v3)
    return out.reshape(B, H, S, Dh)
